# baseline (speedup 1.0000x reference)
import functools

import jax
import jax.numpy as jnp
from jax import lax
from jax.experimental import pallas as pl
from jax.experimental.pallas import tpu as pltpu
from jax.experimental.pallas import tpu_sc as plsc

K = 16
DM = 256
DP = 128
DX = 16


TW = 2 * DM + DP


def _pre_body(points_ref, xyzp_ref, fc1w_ref, fc1b_ref, wk_ref, wv_ref,
              wq_ref, g1_ref, d2_ref, db2_ref,
              T_ref, qg_ref, W2_ref, c2g_ref):
    x = jnp.dot(points_ref[...], fc1w_ref[...]) + fc1b_ref[...]
    g1 = g1_ref[...]
    wkg = jnp.dot(wk_ref[...], g1)
    wqg = jnp.dot(wq_ref[...], g1)
    d2g = jnp.dot(d2_ref[...], g1)
    T_ref[:, 0:DM] = jnp.dot(x, wkg)
    T_ref[:, DM:2 * DM] = jnp.dot(x, wv_ref[...])
    T_ref[:, 2 * DM:TW] = xyzp_ref[...]
    qg_ref[...] = jnp.dot(x, wqg)
    W2_ref[:, 0:DM] = d2_ref[...]
    W2_ref[:, DM:2 * DM] = d2g
    c2g_ref[...] = jnp.dot(db2_ref[...], g1)


def _pre(points2, xyzp128, fc1_w, fc1_b2, wk, wv, wq, gamma_w1, delta_w2,
         delta_b2_2):
    M = points2.shape[0]
    full = lambda shp: pl.BlockSpec(shp, lambda: tuple(0 for _ in shp))
    return pl.pallas_call(
        _pre_body,
        grid=(),
        in_specs=[full((M, DP)), full((M, DP)), full((DP, DM)), full((1, DM)),
                  full((DM, DM)), full((DM, DM)),
                  full((DM, DM)), full((DM, DM)), full((DM, DM)),
                  full((1, DM))],
        out_specs=[full((M, TW)), full((M, DM)), full((DM, 2 * DM)),
                   full((1, DM))],
        out_shape=[jax.ShapeDtypeStruct((M, TW), jnp.float32),
                   jax.ShapeDtypeStruct((M, DM), jnp.float32),
                   jax.ShapeDtypeStruct((DM, 2 * DM), jnp.float32),
                   jax.ShapeDtypeStruct((1, DM), jnp.float32)],
    )(points2, xyzp128, fc1_w, fc1_b2, wk, wv, wq, gamma_w1, delta_w2,
      delta_b2_2)


def _knn_body(n: int, r: int, xyz_ref, xyzT_ref, knn_ref):
    b = pl.program_id(0)
    xi = xyz_ref[0]
    xjT = xyzT_ref[0]
    d0 = xi[:, 0:1] - xjT[0:1, :]
    d = d0 * d0
    d1 = xi[:, 1:2] - xjT[1:2, :]
    d = d + d1 * d1
    d2 = xi[:, 2:3] - xjT[2:3, :]
    d = d + d2 * d2
    iota = lax.broadcasted_iota(jnp.int32, (r, n), 1)
    base = b * n
    ng = n // 128
    cols = []
    for _ in range(K):
        vs = [d[:, g * 128:(g + 1) * 128] for g in range(ng)]
        ix = [iota[:, g * 128:(g + 1) * 128] for g in range(ng)]
        while len(vs) > 1:
            nvs, nix = [], []
            for a in range(0, len(vs), 2):
                c = vs[a] <= vs[a + 1]
                nvs.append(jnp.where(c, vs[a], vs[a + 1]))
                nix.append(jnp.where(c, ix[a], ix[a + 1]))
            vs, ix = nvs, nix
        m = jnp.min(vs[0], axis=1, keepdims=True)
        idx = jnp.min(jnp.where(vs[0] == m, ix[0], n), axis=1)
        cols.append(idx[:, None] + base)
        d = jnp.where(iota == idx[:, None], jnp.inf, d)
    knn_ref[0] = jnp.concatenate(cols, axis=1)


def _knn(xyz, xyzT, r=512):
    B, N, _ = xyz.shape
    return pl.pallas_call(
        functools.partial(_knn_body, N, r),
        grid=(B, N // r),
        in_specs=[pl.BlockSpec((1, r, 3), lambda b, i: (b, i, 0)),
                  pl.BlockSpec((1, 3, N), lambda b, i: (b, 0, 0))],
        out_specs=pl.BlockSpec((1, r, K), lambda b, i: (b, i, 0)),
        out_shape=jax.ShapeDtypeStruct((B, N, K), jnp.int32),
    )(xyz, xyzT)


def _gather_body(rpw: int, ch: int,
                 T_hbm, idx_hbm, G_hbm,
                 idx0, idx1, m0, m1, sm0, sm1):
    wid = lax.axis_index("s") * 2 + lax.axis_index("c")
    base = wid * rpw
    nch = rpw // ch

    def start(c, idxb, mb, sm):
        off = base + c * ch
        pltpu.sync_copy(idx_hbm.at[pl.ds(off, ch)], idxb)
        pltpu.async_copy(T_hbm.at[idxb], mb, sm)

    def finish(c, idxb, mb, sm):
        pltpu.make_async_copy(T_hbm.at[idxb], mb, sm).wait()
        off = base + c * ch
        pltpu.sync_copy(mb, G_hbm.at[pl.ds(off, ch)])

    start(0, idx0, m0, sm0)

    def body(g, carry):
        c0 = g * 2
        start(c0 + 1, idx1, m1, sm1)
        finish(c0, idx0, m0, sm0)

        @pl.when(c0 + 2 < nch)
        def _():
            start(c0 + 2, idx0, m0, sm0)

        finish(c0 + 1, idx1, m1, sm1)
        return carry

    lax.fori_loop(0, nch // 2, body, 0)


def _gather(T, flat_idx, ch=64):
    rows = flat_idx.shape[0]
    nw = 32
    rpw = rows // nw
    mesh = plsc.VectorSubcoreMesh(core_axis_name="c", subcore_axis_name="s")
    fn = pl.kernel(
        functools.partial(_gather_body, rpw, ch),
        out_type=jax.ShapeDtypeStruct((rows, TW), jnp.float32),
        mesh=mesh,
        scratch_types=[pltpu.VMEM((ch,), jnp.int32),
                       pltpu.VMEM((ch,), jnp.int32),
                       pltpu.VMEM((ch, TW), jnp.float32),
                       pltpu.VMEM((ch, TW), jnp.float32),
                       pltpu.SemaphoreType.DMA,
                       pltpu.SemaphoreType.DMA],
    )
    return fn(T, flat_idx)


def _attn_body(nblk: int,
               G_ref, xi_ref, qg_ref, pts_ref, d1p_ref, W2_ref,
               g2_ref, fc2_ref, db1_ref, db2_ref, c2g_ref, gb1_ref,
               gb2_ref, fc2b_ref, attn_ref, res_ref, stA, stP):
    xi = xi_ref[...]
    qg_i = qg_ref[...]
    db1 = db1_ref[...]
    db2 = db2_ref[...]
    c2g = c2g_ref[...]
    gb1 = gb1_ref[...]
    gb2 = gb2_ref[...]
    d1p = d1p_ref[...]
    W2b = W2_ref[...].astype(jnp.bfloat16)
    g2b = (g2_ref[...] * (1.0 / 16.0)).astype(jnp.bfloat16)
    gb2s = gb2 * (1.0 / 16.0)
    qgc = qg_i + gb1 + c2g
    for k in range(K):
        row = G_ref[k]
        kg_j = row[:, 0:DM]
        v_j = row[:, DM:2 * DM]
        rel = xi - row[:, 2 * DM:2 * DM + DX]
        H = jnp.maximum(jnp.dot(rel, d1p) + db1, 0.0)
        PP = jnp.dot(H.astype(jnp.bfloat16), W2b,
                     preferred_element_type=jnp.float32)
        inner = qgc - kg_j + PP[:, DM:2 * DM]
        stA[k] = jnp.dot(jnp.maximum(inner, 0.0).astype(jnp.bfloat16), g2b,
                         preferred_element_type=jnp.float32) + gb2s
        stP[k] = PP[:, 0:DM] + (db2 + v_j)
    m = stA[0]
    for k in range(1, K):
        m = jnp.maximum(m, stA[k])
    s = jnp.zeros_like(m)
    for k in range(K):
        e = jnp.exp(stA[k] - m)
        stA[k] = e
        s = s + e
    rinv = 1.0 / s
    acc = jnp.zeros_like(m)
    for k in range(K):
        a = stA[k] * rinv
        attn_ref[0, :, k, :] = a
        acc = acc + a * stP[k]
    res_ref[...] = (jnp.dot(acc, fc2_ref[...]) + fc2b_ref[...]
                    + pts_ref[...])


def _attn(G3, xyzp, qg, points2, d1p, W2, gamma_w2, fc2_w, db1_2,
          db2_2, c2g, gb1_2, gb2_2, fc2b_2, B, N, p=256):
    nblk = N // p
    wfull = lambda shp: pl.BlockSpec(shp, lambda b, i: tuple(0 for _ in shp))
    return pl.pallas_call(
        functools.partial(_attn_body, nblk),
        grid=(B, nblk),
        in_specs=[
            pl.BlockSpec((K, p, TW), lambda b, i: (0, b * nblk + i, 0)),
            pl.BlockSpec((p, DX), lambda b, i: (b * nblk + i, 0)),
            pl.BlockSpec((p, DM), lambda b, i: (b * nblk + i, 0)),
            pl.BlockSpec((p, DP), lambda b, i: (b * nblk + i, 0)),
            wfull((DX, DM)), wfull((DM, 2 * DM)), wfull((DM, DM)),
            wfull((DM, DP)),
            wfull((1, DM)), wfull((1, DM)), wfull((1, DM)),
            wfull((1, DM)), wfull((1, DM)), wfull((1, DP)),
        ],
        out_specs=[pl.BlockSpec((1, p, K, DM), lambda b, i: (b, i, 0, 0)),
                   pl.BlockSpec((p, DP), lambda b, i: (b * nblk + i, 0))],
        out_shape=[jax.ShapeDtypeStruct((B, N, K, DM), jnp.float32),
                   jax.ShapeDtypeStruct((B * N, DP), jnp.float32)],
        scratch_shapes=[pltpu.VMEM((K, p, DM), jnp.float32),
                        pltpu.VMEM((K, p, DM), jnp.float32)],
    )(G3, xyzp, qg, points2, d1p, W2, gamma_w2, fc2_w, db1_2,
      db2_2, c2g, gb1_2, gb2_2, fc2b_2)


def kernel(xyz, points, fc1_w, fc1_b, fc2_w, fc2_b, delta_w1, delta_b1,
           delta_w2, delta_b2, gamma_w1, gamma_b1, gamma_w2, gamma_b2,
           wq, wk, wv):
    B, N, _ = xyz.shape
    M = B * N
    points2 = points.reshape(M, DP)
    xyz2 = xyz.reshape(M, 3)
    xyzp = jnp.pad(xyz2, ((0, 0), (0, DX - 3)))
    xyzp128 = jnp.pad(xyz2, ((0, 0), (0, DP - 3)))
    d1p = jnp.pad(delta_w1, ((0, DX - 3), (0, 0)))
    xyzT = jnp.swapaxes(xyz, 1, 2)

    T, qg, W2, c2g = _pre(points2, xyzp128, fc1_w, fc1_b.reshape(1, DM),
                          wk, wv, wq, gamma_w1, delta_w2,
                          delta_b2.reshape(1, DM))

    knn_g = _knn(xyz, xyzT)
    flat_idx = jnp.transpose(knn_g, (2, 0, 1)).reshape(K * M)

    G = _gather(T, flat_idx)
    G3 = G.reshape(K, M, TW)

    attn, res2 = _attn(G3, xyzp, qg, points2, d1p, W2, gamma_w2,
                       fc2_w, delta_b1.reshape(1, DM),
                       delta_b2.reshape(1, DM), c2g,
                       gamma_b1.reshape(1, DM), gamma_b2.reshape(1, DM),
                       fc2_b.reshape(1, DP), B, N)
    return (res2.reshape(B, N, DP), attn)


# speedup vs baseline: 1.1804x; 1.1804x over previous
import functools

import jax
import jax.numpy as jnp
from jax import lax
from jax.experimental import pallas as pl
from jax.experimental.pallas import tpu as pltpu
from jax.experimental.pallas import tpu_sc as plsc

K = 16
DM = 256
DP = 128
DX = 16


TW = 2 * DM + DP


def _pre_body(points_ref, xyzp_ref, fc1w_ref, fc1b_ref, wk_ref, wv_ref,
              wq_ref, g1_ref, d2_ref, db2_ref,
              T_ref, qg_ref, W2_ref, c2g_ref):
    x = jnp.dot(points_ref[...], fc1w_ref[...]) + fc1b_ref[...]
    g1 = g1_ref[...]
    wkg = jnp.dot(wk_ref[...], g1)
    wqg = jnp.dot(wq_ref[...], g1)
    d2g = jnp.dot(d2_ref[...], g1)
    T_ref[:, 0:DM] = jnp.dot(x, wkg)
    T_ref[:, DM:2 * DM] = jnp.dot(x, wv_ref[...])
    T_ref[:, 2 * DM:TW] = xyzp_ref[...]
    qg_ref[...] = jnp.dot(x, wqg)
    W2_ref[:, 0:DM] = d2_ref[...]
    W2_ref[:, DM:2 * DM] = d2g
    c2g_ref[...] = jnp.dot(db2_ref[...], g1)


def _pre(points2, xyzp128, fc1_w, fc1_b2, wk, wv, wq, gamma_w1, delta_w2,
         delta_b2_2):
    M = points2.shape[0]
    full = lambda shp: pl.BlockSpec(shp, lambda: tuple(0 for _ in shp))
    return pl.pallas_call(
        _pre_body,
        grid=(),
        in_specs=[full((M, DP)), full((M, DP)), full((DP, DM)), full((1, DM)),
                  full((DM, DM)), full((DM, DM)),
                  full((DM, DM)), full((DM, DM)), full((DM, DM)),
                  full((1, DM))],
        out_specs=[full((M, TW)), full((M, DM)), full((DM, 2 * DM)),
                   full((1, DM))],
        out_shape=[jax.ShapeDtypeStruct((M, TW), jnp.float32),
                   jax.ShapeDtypeStruct((M, DM), jnp.float32),
                   jax.ShapeDtypeStruct((DM, 2 * DM), jnp.float32),
                   jax.ShapeDtypeStruct((1, DM), jnp.float32)],
    )(points2, xyzp128, fc1_w, fc1_b2, wk, wv, wq, gamma_w1, delta_w2,
      delta_b2_2)


def _knn_body(n: int, r: int, base: int, xyz_ref, xyzT_ref, knn_ref):
    xi = xyz_ref[0]
    xjT = xyzT_ref[0]
    d0 = xi[:, 0:1] - xjT[0:1, :]
    d = d0 * d0
    d1 = xi[:, 1:2] - xjT[1:2, :]
    d = d + d1 * d1
    d2 = xi[:, 2:3] - xjT[2:3, :]
    d = d + d2 * d2
    iota = lax.broadcasted_iota(jnp.int32, (r, n), 1)
    cols = []
    for _ in range(K):
        m = jnp.min(d, axis=1, keepdims=True)
        idx = jnp.min(jnp.where(d == m, iota, n), axis=1)
        cols.append(idx[:, None] + base)
        d = jnp.where(iota == idx[:, None], jnp.inf, d)
    knn_ref[0] = jnp.concatenate(cols, axis=1)


def _knn1(xyz_b, xyzT_b, base, r=512):
    _, N, _ = xyz_b.shape
    return pl.pallas_call(
        functools.partial(_knn_body, N, r, base),
        grid=(N // r,),
        in_specs=[pl.BlockSpec((1, r, 3), lambda i: (0, i, 0)),
                  pl.BlockSpec((1, 3, N), lambda i: (0, 0, 0))],
        out_specs=pl.BlockSpec((1, r, K), lambda i: (0, i, 0)),
        out_shape=jax.ShapeDtypeStruct((1, N, K), jnp.int32),
    )(xyz_b, xyzT_b)


def _gather_body(rpw: int, ch: int,
                 T_hbm, idx_hbm, G_hbm,
                 idx0, idx1, m0, m1, sm0, sm1):
    wid = lax.axis_index("s") * 2 + lax.axis_index("c")
    base = wid * rpw
    nch = rpw // ch

    def start(c, idxb, mb, sm):
        off = base + c * ch
        pltpu.sync_copy(idx_hbm.at[pl.ds(off, ch)], idxb)
        pltpu.async_copy(T_hbm.at[idxb], mb, sm)

    def finish(c, idxb, mb, sm):
        pltpu.make_async_copy(T_hbm.at[idxb], mb, sm).wait()
        off = base + c * ch
        pltpu.sync_copy(mb, G_hbm.at[pl.ds(off, ch)])

    start(0, idx0, m0, sm0)

    def body(g, carry):
        c0 = g * 2
        start(c0 + 1, idx1, m1, sm1)
        finish(c0, idx0, m0, sm0)

        @pl.when(c0 + 2 < nch)
        def _():
            start(c0 + 2, idx0, m0, sm0)

        finish(c0 + 1, idx1, m1, sm1)
        return carry

    lax.fori_loop(0, nch // 2, body, 0)


def _gather(T, flat_idx, ch=64):
    rows = flat_idx.shape[0]
    nw = 32
    rpw = rows // nw
    mesh = plsc.VectorSubcoreMesh(core_axis_name="c", subcore_axis_name="s")
    fn = pl.kernel(
        functools.partial(_gather_body, rpw, ch),
        out_type=jax.ShapeDtypeStruct((rows, TW), jnp.float32),
        mesh=mesh,
        scratch_types=[pltpu.VMEM((ch,), jnp.int32),
                       pltpu.VMEM((ch,), jnp.int32),
                       pltpu.VMEM((ch, TW), jnp.float32),
                       pltpu.VMEM((ch, TW), jnp.float32),
                       pltpu.SemaphoreType.DMA,
                       pltpu.SemaphoreType.DMA],
    )
    return fn(T, flat_idx)


def _attn_body(nblk: int,
               G_ref, xi_ref, qg_ref, pts_ref, d1p_ref, W2_ref,
               g2_ref, fc2_ref, db1_ref, db2_ref, c2g_ref, gb1_ref,
               gb2_ref, fc2b_ref, attn_ref, res_ref, stA, stP):
    xi = xi_ref[...]
    qg_i = qg_ref[...]
    db1 = db1_ref[...]
    db2 = db2_ref[...]
    c2g = c2g_ref[...]
    gb1 = gb1_ref[...]
    gb2 = gb2_ref[...]
    d1p = d1p_ref[...]
    W2b = W2_ref[...].astype(jnp.bfloat16)
    g2b = (g2_ref[...] * (1.0 / 16.0)).astype(jnp.bfloat16)
    gb2s = gb2 * (1.0 / 16.0)
    qgc = qg_i + gb1 + c2g
    for k in range(K):
        row = G_ref[k]
        kg_j = row[:, 0:DM]
        v_j = row[:, DM:2 * DM]
        rel = xi - row[:, 2 * DM:2 * DM + DX]
        H = jnp.maximum(jnp.dot(rel, d1p) + db1, 0.0)
        PP = jnp.dot(H.astype(jnp.bfloat16), W2b,
                     preferred_element_type=jnp.float32)
        inner = qgc - kg_j + PP[:, DM:2 * DM]
        stA[k] = jnp.dot(jnp.maximum(inner, 0.0).astype(jnp.bfloat16), g2b,
                         preferred_element_type=jnp.float32) + gb2s
        stP[k] = PP[:, 0:DM] + (db2 + v_j)
    m = stA[0]
    for k in range(1, K):
        m = jnp.maximum(m, stA[k])
    s = jnp.zeros_like(m)
    for k in range(K):
        e = jnp.exp(stA[k] - m)
        stA[k] = e
        s = s + e
    rinv = 1.0 / s
    acc = jnp.zeros_like(m)
    for k in range(K):
        a = stA[k] * rinv
        attn_ref[0, :, k, :] = a
        acc = acc + a * stP[k]
    res_ref[...] = (jnp.dot(acc, fc2_ref[...]) + fc2b_ref[...]
                    + pts_ref[...])


def _attn1(G3, xyzp, qg, points2, d1p, W2, gamma_w2, fc2_w, db1_2,
           db2_2, c2g, gb1_2, gb2_2, fc2b_2, B, N, b0, prev=None, p=256):
    nblk = N // p
    wfull = lambda shp: pl.BlockSpec(shp, lambda i: tuple(0 for _ in shp))
    anyspec = pl.BlockSpec(memory_space=pl.ANY)
    in_specs = [
        pl.BlockSpec((K, p, TW), lambda i: (0, i, 0)),
        pl.BlockSpec((p, DX), lambda i: (b0 * nblk + i, 0)),
        pl.BlockSpec((p, DM), lambda i: (b0 * nblk + i, 0)),
        pl.BlockSpec((p, DP), lambda i: (b0 * nblk + i, 0)),
        wfull((DX, DM)), wfull((DM, 2 * DM)), wfull((DM, DM)),
        wfull((DM, DP)),
        wfull((1, DM)), wfull((1, DM)), wfull((1, DM)),
        wfull((1, DM)), wfull((1, DM)), wfull((1, DP)),
    ]
    args = [G3, xyzp, qg, points2, d1p, W2, gamma_w2, fc2_w, db1_2,
            db2_2, c2g, gb1_2, gb2_2, fc2b_2]
    io_aliases = {}
    if prev is not None:
        in_specs = in_specs + [anyspec, anyspec]
        args = args + [prev[0], prev[1]]
        io_aliases = {14: 0, 15: 1}

    def body(*refs):
        _attn_body(nblk, *refs[:14], refs[-4], refs[-3], refs[-2], refs[-1])

    return pl.pallas_call(
        body,
        grid=(nblk,),
        in_specs=in_specs,
        out_specs=[pl.BlockSpec((1, p, K, DM), lambda i: (b0, i, 0, 0)),
                   pl.BlockSpec((p, DP), lambda i: (b0 * nblk + i, 0))],
        out_shape=[jax.ShapeDtypeStruct((B, N, K, DM), jnp.float32),
                   jax.ShapeDtypeStruct((B * N, DP), jnp.float32)],
        scratch_shapes=[pltpu.VMEM((K, p, DM), jnp.float32),
                        pltpu.VMEM((K, p, DM), jnp.float32)],
        input_output_aliases=io_aliases,
    )(*args)


def kernel(xyz, points, fc1_w, fc1_b, fc2_w, fc2_b, delta_w1, delta_b1,
           delta_w2, delta_b2, gamma_w1, gamma_b1, gamma_w2, gamma_b2,
           wq, wk, wv):
    B, N, _ = xyz.shape
    M = B * N
    points2 = points.reshape(M, DP)
    xyz2 = xyz.reshape(M, 3)
    xyzp = jnp.pad(xyz2, ((0, 0), (0, DX - 3)))
    xyzp128 = jnp.pad(xyz2, ((0, 0), (0, DP - 3)))
    d1p = jnp.pad(delta_w1, ((0, DX - 3), (0, 0)))
    xyzT = jnp.swapaxes(xyz, 1, 2)

    T, qg, W2, c2g = _pre(points2, xyzp128, fc1_w, fc1_b.reshape(1, DM),
                          wk, wv, wq, gamma_w1, delta_w2,
                          delta_b2.reshape(1, DM))

    G3s = []
    for b in range(B):
        knn_b = _knn1(xyz[b:b + 1], xyzT[b:b + 1], b * N)
        flat_b = jnp.transpose(knn_b[0], (1, 0)).reshape(K * N)
        G3s.append(_gather(T, flat_b).reshape(K, N, TW))

    prev = None
    for b in range(B):
        prev = _attn1(G3s[b], xyzp, qg, points2, d1p, W2, gamma_w2,
                      fc2_w, delta_b1.reshape(1, DM),
                      delta_b2.reshape(1, DM), c2g,
                      gamma_b1.reshape(1, DM), gamma_b2.reshape(1, DM),
                      fc2_b.reshape(1, DP), B, N, b, prev)
    attn, res2 = prev
    return (res2.reshape(B, N, DP), attn)


# speedup vs baseline: 1.3289x; 1.1258x over previous
import functools

import jax
import jax.numpy as jnp
from jax import lax
from jax.experimental import pallas as pl
from jax.experimental.pallas import tpu as pltpu
from jax.experimental.pallas import tpu_sc as plsc

K = 16
DM = 256
DP = 128
DX = 16


TW = 3 * DP


def _pack2(a):
    ab = a.astype(jnp.bfloat16)
    lo = lax.bitcast_convert_type(ab[:, 0:DP], jnp.uint16).astype(jnp.uint32)
    hi = lax.bitcast_convert_type(ab[:, DP:DM], jnp.uint16).astype(jnp.uint32)
    return lax.bitcast_convert_type(lo | (hi << 16), jnp.float32)


def _unpack2(w):
    u = lax.bitcast_convert_type(w, jnp.uint32)
    lo = lax.bitcast_convert_type((u & 0xFFFF).astype(jnp.uint16),
                                  jnp.bfloat16)
    hi = lax.bitcast_convert_type((u >> 16).astype(jnp.uint16), jnp.bfloat16)
    return jnp.concatenate([lo, hi], axis=1)


def _pre_body(points_ref, xyzp_ref, fc1w_ref, fc1b_ref, wk_ref, wv_ref,
              wq_ref, g1_ref, d2_ref, db2_ref,
              T_ref, qg_ref, W2_ref, c2g_ref):
    x = jnp.dot(points_ref[...], fc1w_ref[...]) + fc1b_ref[...]
    g1 = g1_ref[...]
    wkg = jnp.dot(wk_ref[...], g1)
    wqg = jnp.dot(wq_ref[...], g1)
    d2g = jnp.dot(d2_ref[...], g1)
    T_ref[:, 0:DP] = _pack2(jnp.dot(x, wkg))
    T_ref[:, DP:2 * DP] = _pack2(jnp.dot(x, wv_ref[...]))
    T_ref[:, 2 * DP:TW] = xyzp_ref[...]
    qg_ref[...] = jnp.dot(x, wqg)
    W2_ref[:, 0:DM] = d2_ref[...]
    W2_ref[:, DM:2 * DM] = d2g
    c2g_ref[...] = jnp.dot(db2_ref[...], g1)


def _pre(points2, xyzp128, fc1_w, fc1_b2, wk, wv, wq, gamma_w1, delta_w2,
         delta_b2_2):
    M = points2.shape[0]
    full = lambda shp: pl.BlockSpec(shp, lambda: tuple(0 for _ in shp))
    return pl.pallas_call(
        _pre_body,
        grid=(),
        in_specs=[full((M, DP)), full((M, DP)), full((DP, DM)), full((1, DM)),
                  full((DM, DM)), full((DM, DM)),
                  full((DM, DM)), full((DM, DM)), full((DM, DM)),
                  full((1, DM))],
        out_specs=[full((M, TW)), full((M, DM)), full((DM, 2 * DM)),
                   full((1, DM))],
        out_shape=[jax.ShapeDtypeStruct((M, TW), jnp.float32),
                   jax.ShapeDtypeStruct((M, DM), jnp.float32),
                   jax.ShapeDtypeStruct((DM, 2 * DM), jnp.float32),
                   jax.ShapeDtypeStruct((1, DM), jnp.float32)],
    )(points2, xyzp128, fc1_w, fc1_b2, wk, wv, wq, gamma_w1, delta_w2,
      delta_b2_2)


def _knn_body(n: int, r: int, base: int, xyz_ref, xyzT_ref, knn_ref):
    xi = xyz_ref[0]
    xjT = xyzT_ref[0]
    d0 = xi[:, 0:1] - xjT[0:1, :]
    d = d0 * d0
    d1 = xi[:, 1:2] - xjT[1:2, :]
    d = d + d1 * d1
    d2 = xi[:, 2:3] - xjT[2:3, :]
    d = d + d2 * d2
    iota = lax.broadcasted_iota(jnp.int32, (r, n), 1)
    cols = []
    for k in range(K):
        m = jnp.min(d, axis=1, keepdims=True)
        idx = jnp.min(jnp.where(d == m, iota, n), axis=1)
        cols.append(idx[:, None] + base)
        if k + 1 < K:
            d = jnp.where(iota == idx[:, None], jnp.inf, d)
    knn_ref[0] = jnp.concatenate(cols, axis=1)


def _knn1(xyz_b, xyzT_b, base, r=512):
    _, N, _ = xyz_b.shape
    return pl.pallas_call(
        functools.partial(_knn_body, N, r, base),
        grid=(N // r,),
        in_specs=[pl.BlockSpec((1, r, 3), lambda i: (0, i, 0)),
                  pl.BlockSpec((1, 3, N), lambda i: (0, 0, 0))],
        out_specs=pl.BlockSpec((1, r, K), lambda i: (0, i, 0)),
        out_shape=jax.ShapeDtypeStruct((1, N, K), jnp.int32),
    )(xyz_b, xyzT_b)


def _gather_body(rpw: int, ch: int,
                 T_hbm, idx_hbm, G_hbm,
                 idx0, idx1, m0, m1, sm0, sm1):
    wid = lax.axis_index("s") * 2 + lax.axis_index("c")
    base = wid * rpw
    nch = rpw // ch

    def start(c, idxb, mb, sm):
        off = base + c * ch
        pltpu.sync_copy(idx_hbm.at[pl.ds(off, ch)], idxb)
        pltpu.async_copy(T_hbm.at[idxb], mb, sm)

    def finish(c, idxb, mb, sm):
        pltpu.make_async_copy(T_hbm.at[idxb], mb, sm).wait()
        off = base + c * ch
        pltpu.sync_copy(mb, G_hbm.at[pl.ds(off, ch)])

    start(0, idx0, m0, sm0)

    def body(g, carry):
        c0 = g * 2
        start(c0 + 1, idx1, m1, sm1)
        finish(c0, idx0, m0, sm0)

        @pl.when(c0 + 2 < nch)
        def _():
            start(c0 + 2, idx0, m0, sm0)

        finish(c0 + 1, idx1, m1, sm1)
        return carry

    lax.fori_loop(0, nch // 2, body, 0)


def _gather(T, flat_idx, ch=64):
    rows = flat_idx.shape[0]
    nw = 32
    rpw = rows // nw
    mesh = plsc.VectorSubcoreMesh(core_axis_name="c", subcore_axis_name="s")
    fn = pl.kernel(
        functools.partial(_gather_body, rpw, ch),
        out_type=jax.ShapeDtypeStruct((rows, TW), jnp.float32),
        mesh=mesh,
        scratch_types=[pltpu.VMEM((ch,), jnp.int32),
                       pltpu.VMEM((ch,), jnp.int32),
                       pltpu.VMEM((ch, TW), jnp.float32),
                       pltpu.VMEM((ch, TW), jnp.float32),
                       pltpu.SemaphoreType.DMA,
                       pltpu.SemaphoreType.DMA],
    )
    return fn(T, flat_idx)


def _attn_body(nblk: int,
               G_ref, xi_ref, qg_ref, pts_ref, d1p_ref, W2_ref,
               g2_ref, fc2_ref, db1_ref, db2_ref, c2g_ref, gb1_ref,
               gb2_ref, fc2b_ref, attn_ref, res_ref, stA, stP):
    xi = xi_ref[...]
    qg_i = qg_ref[...]
    db1 = db1_ref[...]
    db2 = db2_ref[...]
    c2g = c2g_ref[...]
    gb1 = gb1_ref[...]
    gb2 = gb2_ref[...]
    d1p = d1p_ref[...]
    W2b = W2_ref[...].astype(jnp.bfloat16)
    g2b = (g2_ref[...] * (1.0 / 16.0)).astype(jnp.bfloat16)
    gb2s = gb2 * (1.0 / 16.0)
    qgc = (qg_i + gb1 + c2g).astype(jnp.bfloat16)
    for k in range(K):
        row = G_ref[k]
        kg_jb = _unpack2(row[:, 0:DP])
        v_j = _unpack2(row[:, DP:2 * DP]).astype(jnp.float32)
        rel = xi - row[:, 2 * DP:2 * DP + DX]
        H = jnp.maximum(jnp.dot(rel, d1p) + db1, 0.0)
        PP = jnp.dot(H.astype(jnp.bfloat16), W2b,
                     preferred_element_type=jnp.float32)
        inner = qgc - kg_jb + PP[:, DM:2 * DM].astype(jnp.bfloat16)
        stA[k] = jnp.dot(jnp.maximum(inner, jnp.bfloat16(0.0)), g2b,
                         preferred_element_type=jnp.float32) + gb2s
        stP[k] = PP[:, 0:DM] + (db2 + v_j)
    m = stA[0]
    for k in range(1, K):
        m = jnp.maximum(m, stA[k])
    s = jnp.zeros_like(m)
    for k in range(K):
        e = jnp.exp(stA[k] - m)
        stA[k] = e
        s = s + e
    rinv = 1.0 / s
    acc = jnp.zeros_like(m)
    for k in range(K):
        a = stA[k] * rinv
        attn_ref[0, :, k, :] = a
        acc = acc + a * stP[k]
    res_ref[...] = (jnp.dot(acc, fc2_ref[...]) + fc2b_ref[...]
                    + pts_ref[...])


def _attn1(G3, xyzp, qg, points2, d1p, W2, gamma_w2, fc2_w, db1_2,
           db2_2, c2g, gb1_2, gb2_2, fc2b_2, B, N, b0, prev=None, p=256):
    nblk = N // p
    wfull = lambda shp: pl.BlockSpec(shp, lambda i: tuple(0 for _ in shp))
    anyspec = pl.BlockSpec(memory_space=pl.ANY)
    in_specs = [
        pl.BlockSpec((K, p, TW), lambda i: (0, i, 0)),
        pl.BlockSpec((p, DX), lambda i: (b0 * nblk + i, 0)),
        pl.BlockSpec((p, DM), lambda i: (b0 * nblk + i, 0)),
        pl.BlockSpec((p, DP), lambda i: (b0 * nblk + i, 0)),
        wfull((DX, DM)), wfull((DM, 2 * DM)), wfull((DM, DM)),
        wfull((DM, DP)),
        wfull((1, DM)), wfull((1, DM)), wfull((1, DM)),
        wfull((1, DM)), wfull((1, DM)), wfull((1, DP)),
    ]
    args = [G3, xyzp, qg, points2, d1p, W2, gamma_w2, fc2_w, db1_2,
            db2_2, c2g, gb1_2, gb2_2, fc2b_2]
    io_aliases = {}
    if prev is not None:
        in_specs = in_specs + [anyspec, anyspec]
        args = args + [prev[0], prev[1]]
        io_aliases = {14: 0, 15: 1}

    def body(*refs):
        _attn_body(nblk, *refs[:14], refs[-4], refs[-3], refs[-2], refs[-1])

    return pl.pallas_call(
        body,
        grid=(nblk,),
        in_specs=in_specs,
        out_specs=[pl.BlockSpec((1, p, K, DM), lambda i: (b0, i, 0, 0)),
                   pl.BlockSpec((p, DP), lambda i: (b0 * nblk + i, 0))],
        out_shape=[jax.ShapeDtypeStruct((B, N, K, DM), jnp.float32),
                   jax.ShapeDtypeStruct((B * N, DP), jnp.float32)],
        scratch_shapes=[pltpu.VMEM((K, p, DM), jnp.float32),
                        pltpu.VMEM((K, p, DM), jnp.float32)],
        input_output_aliases=io_aliases,
    )(*args)


def kernel(xyz, points, fc1_w, fc1_b, fc2_w, fc2_b, delta_w1, delta_b1,
           delta_w2, delta_b2, gamma_w1, gamma_b1, gamma_w2, gamma_b2,
           wq, wk, wv):
    B, N, _ = xyz.shape
    M = B * N
    points2 = points.reshape(M, DP)
    xyz2 = xyz.reshape(M, 3)
    xyzp = jnp.pad(xyz2, ((0, 0), (0, DX - 3)))
    xyzp128 = jnp.pad(xyz2, ((0, 0), (0, DP - 3)))
    d1p = jnp.pad(delta_w1, ((0, DX - 3), (0, 0)))
    xyzT = jnp.swapaxes(xyz, 1, 2)

    T, qg, W2, c2g = _pre(points2, xyzp128, fc1_w, fc1_b.reshape(1, DM),
                          wk, wv, wq, gamma_w1, delta_w2,
                          delta_b2.reshape(1, DM))

    G3s = []
    for b in range(B):
        knn_b = _knn1(xyz[b:b + 1], xyzT[b:b + 1], b * N)
        flat_b = jnp.transpose(knn_b[0], (1, 0)).reshape(K * N)
        G3s.append(_gather(T, flat_b).reshape(K, N, TW))

    prev = None
    for b in range(B):
        prev = _attn1(G3s[b], xyzp, qg, points2, d1p, W2, gamma_w2,
                      fc2_w, delta_b1.reshape(1, DM),
                      delta_b2.reshape(1, DM), c2g,
                      gamma_b1.reshape(1, DM), gamma_b2.reshape(1, DM),
                      fc2_b.reshape(1, DP), B, N, b, prev)
    attn, res2 = prev
    return (res2.reshape(B, N, DP), attn)


# speedup vs baseline: 1.3995x; 1.0531x over previous
import functools

import jax
import jax.numpy as jnp
from jax import lax
from jax.experimental import pallas as pl
from jax.experimental.pallas import tpu as pltpu
from jax.experimental.pallas import tpu_sc as plsc

K = 16
DM = 256
DP = 128
DX = 16


TW = 3 * DP


def _pack2(a):
    ab = a.astype(jnp.bfloat16)
    lo = lax.bitcast_convert_type(ab[:, 0:DP], jnp.uint16).astype(jnp.uint32)
    hi = lax.bitcast_convert_type(ab[:, DP:DM], jnp.uint16).astype(jnp.uint32)
    return lax.bitcast_convert_type(lo | (hi << 16), jnp.float32)


def _unpack2(w):
    u = lax.bitcast_convert_type(w, jnp.uint32)
    lo = lax.bitcast_convert_type((u & 0xFFFF).astype(jnp.uint16),
                                  jnp.bfloat16)
    hi = lax.bitcast_convert_type((u >> 16).astype(jnp.uint16), jnp.bfloat16)
    return jnp.concatenate([lo, hi], axis=1)


def _pre_body(points_ref, xyzp_ref, fc1w_ref, fc1b_ref, wk_ref, wv_ref,
              wq_ref, g1_ref, d2_ref, db2_ref,
              T_ref, qg_ref, W2_ref, c2g_ref):
    x = jnp.dot(points_ref[...], fc1w_ref[...]) + fc1b_ref[...]
    g1 = g1_ref[...]
    wkg = jnp.dot(wk_ref[...], g1)
    wqg = jnp.dot(wq_ref[...], g1)
    d2g = jnp.dot(d2_ref[...], g1)
    T_ref[:, 0:DP] = _pack2(jnp.dot(x, wkg))
    T_ref[:, DP:2 * DP] = _pack2(jnp.dot(x, wv_ref[...]))
    T_ref[:, 2 * DP:TW] = xyzp_ref[...]
    qg_ref[...] = jnp.dot(x, wqg)
    W2_ref[:, 0:DM] = d2_ref[...]
    W2_ref[:, DM:2 * DM] = d2g
    c2g_ref[...] = jnp.dot(db2_ref[...], g1)


def _pre(points2, xyzp128, fc1_w, fc1_b2, wk, wv, wq, gamma_w1, delta_w2,
         delta_b2_2):
    M = points2.shape[0]
    full = lambda shp: pl.BlockSpec(shp, lambda: tuple(0 for _ in shp))
    return pl.pallas_call(
        _pre_body,
        grid=(),
        in_specs=[full((M, DP)), full((M, DP)), full((DP, DM)), full((1, DM)),
                  full((DM, DM)), full((DM, DM)),
                  full((DM, DM)), full((DM, DM)), full((DM, DM)),
                  full((1, DM))],
        out_specs=[full((M, TW)), full((M, DM)), full((DM, 2 * DM)),
                   full((1, DM))],
        out_shape=[jax.ShapeDtypeStruct((M, TW), jnp.float32),
                   jax.ShapeDtypeStruct((M, DM), jnp.float32),
                   jax.ShapeDtypeStruct((DM, 2 * DM), jnp.float32),
                   jax.ShapeDtypeStruct((1, DM), jnp.float32)],
    )(points2, xyzp128, fc1_w, fc1_b2, wk, wv, wq, gamma_w1, delta_w2,
      delta_b2_2)


def _knn_body(n: int, r: int, base: int, xyz_ref, xyzT_ref, knn_ref):
    xi = xyz_ref[0]
    xjT = xyzT_ref[0]
    d0 = xi[:, 0:1] - xjT[0:1, :]
    d = d0 * d0
    d1 = xi[:, 1:2] - xjT[1:2, :]
    d = d + d1 * d1
    d2 = xi[:, 2:3] - xjT[2:3, :]
    d = d + d2 * d2
    iota = lax.broadcasted_iota(jnp.int32, (r, n), 1)
    iota128 = lax.broadcasted_iota(jnp.int32, (r, 128), 1)
    ng = n // 128
    cols = []
    for k in range(K):
        v = d[:, 0:128]
        gw = jnp.zeros((r, 128), jnp.int32)
        for g in range(1, ng):
            dg = d[:, g * 128:(g + 1) * 128]
            c = v <= dg
            v = jnp.where(c, v, dg)
            gw = jnp.where(c, gw, jnp.int32(g))
        gidx = gw * 128 + iota128
        m = jnp.min(v, axis=1, keepdims=True)
        idx = jnp.min(jnp.where(v == m, gidx, n), axis=1)
        cols.append(idx[:, None] + base)
        if k + 1 < K:
            d = jnp.where(iota == idx[:, None], jnp.inf, d)
    knn = jnp.concatenate(cols, axis=1)
    knn_ref[...] = jnp.transpose(knn, (1, 0))


def _knn1(xyz_b, xyzT_b, base, r=512):
    _, N, _ = xyz_b.shape
    return pl.pallas_call(
        functools.partial(_knn_body, N, r, base),
        grid=(N // r,),
        in_specs=[pl.BlockSpec((1, r, 3), lambda i: (0, i, 0)),
                  pl.BlockSpec((1, 3, N), lambda i: (0, 0, 0))],
        out_specs=pl.BlockSpec((K, r), lambda i: (0, i)),
        out_shape=jax.ShapeDtypeStruct((K, N), jnp.int32),
    )(xyz_b, xyzT_b)


def _gather_body(rpw: int, ch: int,
                 T_hbm, idx_hbm, G_hbm,
                 idx0, idx1, m0, m1, sm0, sm1):
    wid = lax.axis_index("s") * 2 + lax.axis_index("c")
    base = wid * rpw
    nch = rpw // ch

    def start(c, idxb, mb, sm):
        off = base + c * ch
        pltpu.sync_copy(idx_hbm.at[pl.ds(off, ch)], idxb)
        pltpu.async_copy(T_hbm.at[idxb], mb, sm)

    def finish(c, idxb, mb, sm):
        pltpu.make_async_copy(T_hbm.at[idxb], mb, sm).wait()
        off = base + c * ch
        pltpu.sync_copy(mb, G_hbm.at[pl.ds(off, ch)])

    start(0, idx0, m0, sm0)

    def body(g, carry):
        c0 = g * 2
        start(c0 + 1, idx1, m1, sm1)
        finish(c0, idx0, m0, sm0)

        @pl.when(c0 + 2 < nch)
        def _():
            start(c0 + 2, idx0, m0, sm0)

        finish(c0 + 1, idx1, m1, sm1)
        return carry

    lax.fori_loop(0, nch // 2, body, 0)


def _gather(T, flat_idx, ch=64):
    rows = flat_idx.shape[0]
    nw = 32
    rpw = rows // nw
    mesh = plsc.VectorSubcoreMesh(core_axis_name="c", subcore_axis_name="s")
    fn = pl.kernel(
        functools.partial(_gather_body, rpw, ch),
        out_type=jax.ShapeDtypeStruct((rows, TW), jnp.float32),
        mesh=mesh,
        scratch_types=[pltpu.VMEM((ch,), jnp.int32),
                       pltpu.VMEM((ch,), jnp.int32),
                       pltpu.VMEM((ch, TW), jnp.float32),
                       pltpu.VMEM((ch, TW), jnp.float32),
                       pltpu.SemaphoreType.DMA,
                       pltpu.SemaphoreType.DMA],
    )
    return fn(T, flat_idx)


def _attn_body(nblk: int,
               G_ref, xi_ref, qg_ref, pts_ref, d1p_ref, W2_ref,
               g2_ref, fc2_ref, db1_ref, db2_ref, c2g_ref, gb1_ref,
               gb2_ref, fc2b_ref, attn_ref, res_ref, stA, stP):
    xi = xi_ref[...]
    qg_i = qg_ref[...]
    db1 = db1_ref[...]
    db2 = db2_ref[...]
    c2g = c2g_ref[...]
    gb1 = gb1_ref[...]
    gb2 = gb2_ref[...]
    d1p = d1p_ref[...]
    W2b = W2_ref[...].astype(jnp.bfloat16)
    g2b = (g2_ref[...] * (1.0 / 16.0)).astype(jnp.bfloat16)
    gb2s = gb2 * (1.0 / 16.0)
    qgc = (qg_i + gb1 + c2g).astype(jnp.bfloat16)
    for k in range(K):
        row = G_ref[k]
        kg_jb = _unpack2(row[:, 0:DP])
        v_j = _unpack2(row[:, DP:2 * DP]).astype(jnp.float32)
        rel = xi - row[:, 2 * DP:2 * DP + DX]
        H = jnp.maximum(jnp.dot(rel, d1p) + db1, 0.0)
        PP = jnp.dot(H.astype(jnp.bfloat16), W2b,
                     preferred_element_type=jnp.float32)
        inner = qgc - kg_jb + PP[:, DM:2 * DM].astype(jnp.bfloat16)
        stA[k] = jnp.dot(jnp.maximum(inner, jnp.bfloat16(0.0)), g2b,
                         preferred_element_type=jnp.float32) + gb2s
        stP[k] = PP[:, 0:DM] + (db2 + v_j)
    m = stA[0]
    for k in range(1, K):
        m = jnp.maximum(m, stA[k])
    s = jnp.zeros_like(m)
    for k in range(K):
        e = jnp.exp(stA[k] - m)
        stA[k] = e
        s = s + e
    rinv = 1.0 / s
    acc = jnp.zeros_like(m)
    for k in range(K):
        a = stA[k] * rinv
        attn_ref[0, :, k, :] = a
        acc = acc + a * stP[k]
    res_ref[...] = (jnp.dot(acc, fc2_ref[...]) + fc2b_ref[...]
                    + pts_ref[...])


def _attn1(G3, xyzp, qg, points2, d1p, W2, gamma_w2, fc2_w, db1_2,
           db2_2, c2g, gb1_2, gb2_2, fc2b_2, B, N, b0, prev=None, p=256):
    nblk = N // p
    wfull = lambda shp: pl.BlockSpec(shp, lambda i: tuple(0 for _ in shp))
    anyspec = pl.BlockSpec(memory_space=pl.ANY)
    in_specs = [
        pl.BlockSpec((K, p, TW), lambda i: (0, i, 0)),
        pl.BlockSpec((p, DX), lambda i: (b0 * nblk + i, 0)),
        pl.BlockSpec((p, DM), lambda i: (b0 * nblk + i, 0)),
        pl.BlockSpec((p, DP), lambda i: (b0 * nblk + i, 0)),
        wfull((DX, DM)), wfull((DM, 2 * DM)), wfull((DM, DM)),
        wfull((DM, DP)),
        wfull((1, DM)), wfull((1, DM)), wfull((1, DM)),
        wfull((1, DM)), wfull((1, DM)), wfull((1, DP)),
    ]
    args = [G3, xyzp, qg, points2, d1p, W2, gamma_w2, fc2_w, db1_2,
            db2_2, c2g, gb1_2, gb2_2, fc2b_2]
    io_aliases = {}
    if prev is not None:
        in_specs = in_specs + [anyspec, anyspec]
        args = args + [prev[0], prev[1]]
        io_aliases = {14: 0, 15: 1}

    def body(*refs):
        _attn_body(nblk, *refs[:14], refs[-4], refs[-3], refs[-2], refs[-1])

    return pl.pallas_call(
        body,
        grid=(nblk,),
        in_specs=in_specs,
        out_specs=[pl.BlockSpec((1, p, K, DM), lambda i: (b0, i, 0, 0)),
                   pl.BlockSpec((p, DP), lambda i: (b0 * nblk + i, 0))],
        out_shape=[jax.ShapeDtypeStruct((B, N, K, DM), jnp.float32),
                   jax.ShapeDtypeStruct((B * N, DP), jnp.float32)],
        scratch_shapes=[pltpu.VMEM((K, p, DM), jnp.float32),
                        pltpu.VMEM((K, p, DM), jnp.float32)],
        input_output_aliases=io_aliases,
    )(*args)


def kernel(xyz, points, fc1_w, fc1_b, fc2_w, fc2_b, delta_w1, delta_b1,
           delta_w2, delta_b2, gamma_w1, gamma_b1, gamma_w2, gamma_b2,
           wq, wk, wv):
    B, N, _ = xyz.shape
    M = B * N
    points2 = points.reshape(M, DP)
    xyz2 = xyz.reshape(M, 3)
    xyzp = jnp.pad(xyz2, ((0, 0), (0, DX - 3)))
    xyzp128 = jnp.pad(xyz2, ((0, 0), (0, DP - 3)))
    d1p = jnp.pad(delta_w1, ((0, DX - 3), (0, 0)))
    xyzT = jnp.swapaxes(xyz, 1, 2)

    T, qg, W2, c2g = _pre(points2, xyzp128, fc1_w, fc1_b.reshape(1, DM),
                          wk, wv, wq, gamma_w1, delta_w2,
                          delta_b2.reshape(1, DM))

    G3s = []
    for b in range(B):
        knn_b = _knn1(xyz[b:b + 1], xyzT[b:b + 1], b * N)
        G3s.append(_gather(T, knn_b.reshape(K * N)).reshape(K, N, TW))

    prev = None
    for b in range(B):
        prev = _attn1(G3s[b], xyzp, qg, points2, d1p, W2, gamma_w2,
                      fc2_w, delta_b1.reshape(1, DM),
                      delta_b2.reshape(1, DM), c2g,
                      gamma_b1.reshape(1, DM), gamma_b2.reshape(1, DM),
                      fc2_b.reshape(1, DP), B, N, b, prev)
    attn, res2 = prev
    return (res2.reshape(B, N, DP), attn)


# speedup vs baseline: 1.4084x; 1.0064x over previous
import functools

import jax
import jax.numpy as jnp
from jax import lax
from jax.experimental import pallas as pl
from jax.experimental.pallas import tpu as pltpu
from jax.experimental.pallas import tpu_sc as plsc

K = 16
DM = 256
DP = 128
DX = 16


TW = 3 * DP


def _pack2(a):
    ab = a.astype(jnp.bfloat16)
    lo = lax.bitcast_convert_type(ab[:, 0:DP], jnp.uint16).astype(jnp.uint32)
    hi = lax.bitcast_convert_type(ab[:, DP:DM], jnp.uint16).astype(jnp.uint32)
    return lax.bitcast_convert_type(lo | (hi << 16), jnp.float32)


def _unpack2(w):
    u = lax.bitcast_convert_type(w, jnp.uint32)
    lo = lax.bitcast_convert_type((u & 0xFFFF).astype(jnp.uint16),
                                  jnp.bfloat16)
    hi = lax.bitcast_convert_type((u >> 16).astype(jnp.uint16), jnp.bfloat16)
    return jnp.concatenate([lo, hi], axis=1)


def _pre_body(points_ref, xyzp_ref, fc1w_ref, fc1b_ref, wk_ref, wv_ref,
              wq_ref, g1_ref, d2_ref, db2_ref,
              T_ref, qg_ref, W2_ref, c2g_ref):
    x = jnp.dot(points_ref[...], fc1w_ref[...]) + fc1b_ref[...]
    g1 = g1_ref[...]
    wkg = jnp.dot(wk_ref[...], g1)
    wqg = jnp.dot(wq_ref[...], g1)
    d2g = jnp.dot(d2_ref[...], g1)
    T_ref[:, 0:DP] = _pack2(jnp.dot(x, wkg))
    T_ref[:, DP:2 * DP] = _pack2(jnp.dot(x, wv_ref[...]))
    T_ref[:, 2 * DP:TW] = xyzp_ref[...]
    qg_ref[...] = jnp.dot(x, wqg)
    W2_ref[:, 0:DM] = d2_ref[...]
    W2_ref[:, DM:2 * DM] = d2g
    c2g_ref[...] = jnp.dot(db2_ref[...], g1)


def _pre(points2, xyzp128, fc1_w, fc1_b2, wk, wv, wq, gamma_w1, delta_w2,
         delta_b2_2):
    M = points2.shape[0]
    full = lambda shp: pl.BlockSpec(shp, lambda: tuple(0 for _ in shp))
    return pl.pallas_call(
        _pre_body,
        grid=(),
        in_specs=[full((M, DP)), full((M, DP)), full((DP, DM)), full((1, DM)),
                  full((DM, DM)), full((DM, DM)),
                  full((DM, DM)), full((DM, DM)), full((DM, DM)),
                  full((1, DM))],
        out_specs=[full((M, TW)), full((M, DM)), full((DM, 2 * DM)),
                   full((1, DM))],
        out_shape=[jax.ShapeDtypeStruct((M, TW), jnp.float32),
                   jax.ShapeDtypeStruct((M, DM), jnp.float32),
                   jax.ShapeDtypeStruct((DM, 2 * DM), jnp.float32),
                   jax.ShapeDtypeStruct((1, DM), jnp.float32)],
    )(points2, xyzp128, fc1_w, fc1_b2, wk, wv, wq, gamma_w1, delta_w2,
      delta_b2_2)


def _knn_body(n: int, r: int, base: int, xyz_ref, xyzT_ref, knn_ref):
    xi = xyz_ref[0]
    xjT = xyzT_ref[0]
    d0 = xi[:, 0:1] - xjT[0:1, :]
    d = d0 * d0
    d1 = xi[:, 1:2] - xjT[1:2, :]
    d = d + d1 * d1
    d2 = xi[:, 2:3] - xjT[2:3, :]
    d = d + d2 * d2
    iota = lax.broadcasted_iota(jnp.int32, (r, n), 1)
    iota128 = lax.broadcasted_iota(jnp.int32, (r, 128), 1)
    ng = n // 128
    cols = []
    for k in range(K):
        v = d[:, 0:128]
        gw = jnp.zeros((r, 128), jnp.int32)
        for g in range(1, ng):
            dg = d[:, g * 128:(g + 1) * 128]
            c = v <= dg
            v = jnp.where(c, v, dg)
            gw = jnp.where(c, gw, jnp.int32(g))
        gidx = gw * 128 + iota128
        m = jnp.min(v, axis=1, keepdims=True)
        idx = jnp.min(jnp.where(v == m, gidx, n), axis=1)
        cols.append(idx[:, None] + base)
        if k + 1 < K:
            d = jnp.where(iota == idx[:, None], jnp.inf, d)
    knn = jnp.concatenate(cols, axis=1)
    knn_ref[...] = jnp.transpose(knn, (1, 0))


def _knn1(xyz_b, xyzT_b, base, r=512):
    _, N, _ = xyz_b.shape
    return pl.pallas_call(
        functools.partial(_knn_body, N, r, base),
        grid=(N // r,),
        in_specs=[pl.BlockSpec((1, r, 3), lambda i: (0, i, 0)),
                  pl.BlockSpec((1, 3, N), lambda i: (0, 0, 0))],
        out_specs=pl.BlockSpec((K, r), lambda i: (0, i)),
        out_shape=jax.ShapeDtypeStruct((K, N), jnp.int32),
    )(xyz_b, xyzT_b)


def _gather_body(rpw: int, ch: int,
                 T_hbm, idx_hbm, G_hbm,
                 idx0, idx1, m0, m1, sm0, sm1):
    wid = lax.axis_index("s") * 2 + lax.axis_index("c")
    base = wid * rpw
    nch = rpw // ch

    def start(c, idxb, mb, sm):
        off = base + c * ch
        pltpu.sync_copy(idx_hbm.at[pl.ds(off, ch)], idxb)
        pltpu.async_copy(T_hbm.at[idxb], mb, sm)

    def finish(c, idxb, mb, sm):
        pltpu.make_async_copy(T_hbm.at[idxb], mb, sm).wait()
        off = base + c * ch
        pltpu.sync_copy(mb, G_hbm.at[pl.ds(off, ch)])

    start(0, idx0, m0, sm0)

    def body(g, carry):
        c0 = g * 2
        start(c0 + 1, idx1, m1, sm1)
        finish(c0, idx0, m0, sm0)

        @pl.when(c0 + 2 < nch)
        def _():
            start(c0 + 2, idx0, m0, sm0)

        finish(c0 + 1, idx1, m1, sm1)
        return carry

    lax.fori_loop(0, nch // 2, body, 0)


def _gather(T, flat_idx, ch=128):
    rows = flat_idx.shape[0]
    nw = 32
    rpw = rows // nw
    mesh = plsc.VectorSubcoreMesh(core_axis_name="c", subcore_axis_name="s")
    fn = pl.kernel(
        functools.partial(_gather_body, rpw, ch),
        out_type=jax.ShapeDtypeStruct((rows, TW), jnp.float32),
        mesh=mesh,
        scratch_types=[pltpu.VMEM((ch,), jnp.int32),
                       pltpu.VMEM((ch,), jnp.int32),
                       pltpu.VMEM((ch, TW), jnp.float32),
                       pltpu.VMEM((ch, TW), jnp.float32),
                       pltpu.SemaphoreType.DMA,
                       pltpu.SemaphoreType.DMA],
    )
    return fn(T, flat_idx)


def _attn_body(nblk: int,
               G_ref, xi_ref, qg_ref, pts_ref, d1p_ref, W2_ref,
               g2_ref, fc2_ref, db1_ref, db2_ref, c2g_ref, gb1_ref,
               gb2_ref, fc2b_ref, attn_ref, res_ref, stA, stP):
    xi = xi_ref[...]
    qg_i = qg_ref[...]
    db1 = db1_ref[...]
    db2 = db2_ref[...]
    c2g = c2g_ref[...]
    gb1 = gb1_ref[...]
    gb2 = gb2_ref[...]
    d1p = d1p_ref[...]
    W2b = W2_ref[...].astype(jnp.bfloat16)
    g2b = (g2_ref[...] * (1.0 / 16.0)).astype(jnp.bfloat16)
    gb2s = gb2 * (1.0 / 16.0)
    qgc = (qg_i + gb1 + c2g).astype(jnp.bfloat16)
    for k in range(K):
        row = G_ref[k]
        kg_jb = _unpack2(row[:, 0:DP])
        v_j = _unpack2(row[:, DP:2 * DP]).astype(jnp.float32)
        rel = xi - row[:, 2 * DP:2 * DP + DX]
        H = jnp.maximum(jnp.dot(rel, d1p) + db1, 0.0)
        PP = jnp.dot(H.astype(jnp.bfloat16), W2b,
                     preferred_element_type=jnp.float32)
        inner = qgc - kg_jb + PP[:, DM:2 * DM].astype(jnp.bfloat16)
        stA[k] = jnp.dot(jnp.maximum(inner, jnp.bfloat16(0.0)), g2b,
                         preferred_element_type=jnp.float32) + gb2s
        stP[k] = PP[:, 0:DM] + (db2 + v_j)
    m = stA[0]
    for k in range(1, K):
        m = jnp.maximum(m, stA[k])
    s = jnp.zeros_like(m)
    for k in range(K):
        e = jnp.exp(stA[k] - m)
        stA[k] = e
        s = s + e
    rinv = 1.0 / s
    acc = jnp.zeros_like(m)
    for k in range(K):
        a = stA[k] * rinv
        attn_ref[0, :, k, :] = a
        acc = acc + a * stP[k]
    res_ref[...] = (jnp.dot(acc, fc2_ref[...]) + fc2b_ref[...]
                    + pts_ref[...])


def _attn1(G3, xyzp, qg, points2, d1p, W2, gamma_w2, fc2_w, db1_2,
           db2_2, c2g, gb1_2, gb2_2, fc2b_2, B, N, b0, prev=None, p=256):
    nblk = N // p
    wfull = lambda shp: pl.BlockSpec(shp, lambda i: tuple(0 for _ in shp))
    anyspec = pl.BlockSpec(memory_space=pl.ANY)
    in_specs = [
        pl.BlockSpec((K, p, TW), lambda i: (0, i, 0)),
        pl.BlockSpec((p, DX), lambda i: (b0 * nblk + i, 0)),
        pl.BlockSpec((p, DM), lambda i: (b0 * nblk + i, 0)),
        pl.BlockSpec((p, DP), lambda i: (b0 * nblk + i, 0)),
        wfull((DX, DM)), wfull((DM, 2 * DM)), wfull((DM, DM)),
        wfull((DM, DP)),
        wfull((1, DM)), wfull((1, DM)), wfull((1, DM)),
        wfull((1, DM)), wfull((1, DM)), wfull((1, DP)),
    ]
    args = [G3, xyzp, qg, points2, d1p, W2, gamma_w2, fc2_w, db1_2,
            db2_2, c2g, gb1_2, gb2_2, fc2b_2]
    io_aliases = {}
    if prev is not None:
        in_specs = in_specs + [anyspec, anyspec]
        args = args + [prev[0], prev[1]]
        io_aliases = {14: 0, 15: 1}

    def body(*refs):
        _attn_body(nblk, *refs[:14], refs[-4], refs[-3], refs[-2], refs[-1])

    return pl.pallas_call(
        body,
        grid=(nblk,),
        in_specs=in_specs,
        out_specs=[pl.BlockSpec((1, p, K, DM), lambda i: (b0, i, 0, 0)),
                   pl.BlockSpec((p, DP), lambda i: (b0 * nblk + i, 0))],
        out_shape=[jax.ShapeDtypeStruct((B, N, K, DM), jnp.float32),
                   jax.ShapeDtypeStruct((B * N, DP), jnp.float32)],
        scratch_shapes=[pltpu.VMEM((K, p, DM), jnp.float32),
                        pltpu.VMEM((K, p, DM), jnp.float32)],
        input_output_aliases=io_aliases,
    )(*args)


def kernel(xyz, points, fc1_w, fc1_b, fc2_w, fc2_b, delta_w1, delta_b1,
           delta_w2, delta_b2, gamma_w1, gamma_b1, gamma_w2, gamma_b2,
           wq, wk, wv):
    B, N, _ = xyz.shape
    M = B * N
    points2 = points.reshape(M, DP)
    xyz2 = xyz.reshape(M, 3)
    xyzp = jnp.pad(xyz2, ((0, 0), (0, DX - 3)))
    xyzp128 = jnp.pad(xyz2, ((0, 0), (0, DP - 3)))
    d1p = jnp.pad(delta_w1, ((0, DX - 3), (0, 0)))
    xyzT = jnp.swapaxes(xyz, 1, 2)

    T, qg, W2, c2g = _pre(points2, xyzp128, fc1_w, fc1_b.reshape(1, DM),
                          wk, wv, wq, gamma_w1, delta_w2,
                          delta_b2.reshape(1, DM))

    G3s = []
    for b in range(B):
        knn_b = _knn1(xyz[b:b + 1], xyzT[b:b + 1], b * N)
        G3s.append(_gather(T, knn_b.reshape(K * N)).reshape(K, N, TW))

    prev = None
    for b in range(B):
        prev = _attn1(G3s[b], xyzp, qg, points2, d1p, W2, gamma_w2,
                      fc2_w, delta_b1.reshape(1, DM),
                      delta_b2.reshape(1, DM), c2g,
                      gamma_b1.reshape(1, DM), gamma_b2.reshape(1, DM),
                      fc2_b.reshape(1, DP), B, N, b, prev)
    attn, res2 = prev
    return (res2.reshape(B, N, DP), attn)
